# baseline (speedup 1.0000x reference)
import functools

import jax
import jax.numpy as jnp
from jax import lax
from jax.experimental import pallas as pl
from jax.experimental.pallas import tpu as pltpu
from jax.experimental.pallas import tpu_sc as plsc

B, S, H = 4096, 200, 64
N = B * S
NC, NS = 2, 16
NW = NC * NS
PER_W = N // NW
CHUNK = 800
N_CHUNKS = PER_W // CHUNK
REP = CHUNK // S
LANES = 16

_mesh = plsc.VectorSubcoreMesh(core_axis_name="c", subcore_axis_name="s")


@functools.partial(
    pl.kernel,
    out_type=jax.ShapeDtypeStruct((N, H), jnp.float32),
    mesh=_mesh,
    scratch_types=[
        pltpu.VMEM((CHUNK,), jnp.int32),
        pltpu.VMEM((CHUNK, H), jnp.float32),
        pltpu.VMEM((CHUNK, H), jnp.float32),
        pltpu.SemaphoreType.DMA,
    ],
    compiler_params=pltpu.CompilerParams(use_tc_tiling_on_sc=False),
)
def _emb_kernel(ids_hbm, table_hbm, pos_hbm, out_hbm, idx_v, rows_v, pos_rep, sem):
    wid = lax.axis_index("s") * NC + lax.axis_index("c")
    base_w = wid * PER_W

    for q in range(REP):
        pltpu.sync_copy(pos_hbm.at[pl.ds(0, S)], pos_rep.at[pl.ds(q * S, S)])

    def chunk_body(c, carry):
        base = base_w + c * CHUNK
        pltpu.sync_copy(ids_hbm.at[pl.ds(base, CHUNK)], idx_v)
        pltpu.async_copy(table_hbm.at[idx_v], rows_v, sem).wait()

        def add_body(i, carry2):
            for k in range(H // LANES):
                sl = pl.ds(k * LANES, LANES)
                rows_v[i, sl] = rows_v[i, sl] + pos_rep[i, sl]
            return carry2

        lax.fori_loop(0, CHUNK, add_body, 0)
        pltpu.sync_copy(rows_v, out_hbm.at[pl.ds(base, CHUNK)])
        return carry

    lax.fori_loop(0, N_CHUNKS, chunk_body, 0)


def kernel(input_ids, word_table, pos_table):
    ids_flat = input_ids.reshape(-1).astype(jnp.int32)
    out = _emb_kernel(ids_flat, word_table, pos_table)
    return out.reshape(B, S, H)
